# Initial kernel scaffold; baseline (speedup 1.0000x reference)
import jax
import jax.numpy as jnp
from jax.experimental import pallas as pl


def kernel(x, edge_index, W1, b1, W_mu, b_mu, W_ls, b_ls):
    raise NotImplementedError("write your pallas kernel here")


# speedup vs baseline: 28.8412x; 28.8412x over previous
import functools

import jax
import jax.numpy as jnp
from jax import lax
from jax.experimental import pallas as pl
from jax.experimental.pallas import tpu as pltpu
from jax.experimental.pallas import tpu_sc as plsc

N_NODES_K = 10000
N_PAD = 10240
TRASH = N_NODES_K
NC = 2
NS = 16
NW = NC * NS
CH = 128
E_EDGES = 320000
K_PW = -(-E_EDGES // (NW * CH))
E_PAD = NW * CH * K_PW
STRIPE = N_PAD // NS

_MESH = plsc.VectorSubcoreMesh(
    core_axis_name="c", subcore_axis_name="s", num_cores=NC, num_subcores=NS)


def _make_deg_kernel():
    @functools.partial(
        pl.kernel,
        out_type=jax.ShapeDtypeStruct((NC, N_PAD, 8), jnp.float32),
        mesh=_MESH,
        scratch_types=[
            pltpu.VMEM((K_PW, CH), jnp.int32),
            pltpu.VMEM((CH, 8), jnp.float32),
            pltpu.VMEM_SHARED((N_PAD, 8), jnp.float32),
        ],
        compiler_params=pltpu.CompilerParams(use_tc_tiling_on_sc=False),
    )
    def deg_kernel(dst_hbm, ones_hbm, zeros_hbm, out_hbm, dst_v, ones_v, acc):
        cid = lax.axis_index("c")
        sid = lax.axis_index("s")
        wid = cid * NS + sid
        rows = pl.ds(sid * STRIPE, STRIPE)
        pltpu.sync_copy(zeros_hbm, acc.at[rows])
        pltpu.sync_copy(dst_hbm.at[wid], dst_v)
        pltpu.sync_copy(ones_hbm, ones_v)
        plsc.subcore_barrier()

        def body(j, carry):
            pltpu.sync_copy(ones_v, acc.at[dst_v.at[j]], add=True)
            return carry

        lax.fori_loop(0, K_PW, body, 0)
        plsc.subcore_barrier()
        pltpu.sync_copy(acc.at[rows], out_hbm.at[cid, rows])

    return deg_kernel


def _make_agg_kernel():
    @functools.partial(
        pl.kernel,
        out_type=jax.ShapeDtypeStruct((NC, N_PAD, 32), jnp.float32),
        mesh=_MESH,
        scratch_types=[
            pltpu.VMEM((K_PW, CH), jnp.int32),
            pltpu.VMEM((K_PW, CH), jnp.int32),
            pltpu.VMEM((CH, 32), jnp.float32),
            pltpu.VMEM_SHARED((N_PAD, 32), jnp.float32),
            pltpu.SemaphoreType.DMA,
        ],
        compiler_params=pltpu.CompilerParams(use_tc_tiling_on_sc=False),
    )
    def agg_kernel(table_hbm, src_hbm, dst_hbm, zeros_hbm, out_hbm,
                   src_v, dst_v, rows_v, acc, sem):
        cid = lax.axis_index("c")
        sid = lax.axis_index("s")
        wid = cid * NS + sid
        rows = pl.ds(sid * STRIPE, STRIPE)
        pltpu.sync_copy(zeros_hbm, acc.at[rows])
        pltpu.sync_copy(src_hbm.at[wid], src_v)
        pltpu.sync_copy(dst_hbm.at[wid], dst_v)
        plsc.subcore_barrier()

        def body(j, carry):
            pltpu.async_copy(table_hbm.at[src_v.at[j]], rows_v, sem).wait()
            pltpu.sync_copy(rows_v, acc.at[dst_v.at[j]], add=True)
            return carry

        lax.fori_loop(0, K_PW, body, 0)
        plsc.subcore_barrier()
        pltpu.sync_copy(acc.at[rows], out_hbm.at[cid, rows])

    return agg_kernel


_R = 2048


def _tc1_body(x_ref, w_ref, d0_ref, d1_ref, out_ref):
    deg = d0_ref[:, 0:1] + d1_ref[:, 0:1] + 1.0
    dinv = lax.rsqrt(deg)
    t = jnp.dot(x_ref[...], w_ref[...], preferred_element_type=jnp.float32)
    out_ref[...] = t * dinv


def _tc2_body(a0_ref, a1_ref, ts_ref, d0_ref, d1_ref, b1_ref, out_ref):
    deg = d0_ref[:, 0:1] + d1_ref[:, 0:1] + 1.0
    dinv = lax.rsqrt(deg)
    h = dinv * (a0_ref[...] + a1_ref[...] + ts_ref[...]) + b1_ref[...]
    out_ref[...] = jnp.maximum(h, 0.0) * dinv


def _tc3_body(a0_ref, a1_ref, hs_ref, d0_ref, d1_ref,
              wm_ref, bm_ref, wl_ref, bl_ref, mu_ref, ls_ref):
    deg = d0_ref[:, 0:1] + d1_ref[:, 0:1] + 1.0
    dinv = lax.rsqrt(deg)
    p = dinv * (a0_ref[...] + a1_ref[...] + hs_ref[...])
    mu_ref[...] = jnp.dot(p, wm_ref[...], preferred_element_type=jnp.float32) + bm_ref[...]
    ls_ref[...] = jnp.dot(p, wl_ref[...], preferred_element_type=jnp.float32) + bl_ref[...]


def _row_spec(w):
    return pl.BlockSpec((_R, w), lambda i: (i, 0))


def _full_spec(shape):
    return pl.BlockSpec(shape, lambda i: (0,) * len(shape))


def kernel(x, edge_index, W1, b1, W_mu, b_mu, W_ls, b_ls):
    src = edge_index[0].astype(jnp.int32)
    dst = edge_index[1].astype(jnp.int32)
    npad_e = E_PAD - E_EDGES
    pad_idx = jnp.full((npad_e,), TRASH, dtype=jnp.int32)
    src_p = jnp.concatenate([src, pad_idx]).reshape(NW, K_PW, CH)
    dst_p = jnp.concatenate([dst, pad_idx]).reshape(NW, K_PW, CH)

    x_pad = jnp.pad(x, ((0, N_PAD - N_NODES_K), (0, 0)))
    ones8 = jnp.ones((CH, 8), dtype=jnp.float32)
    zeros8 = jnp.zeros((STRIPE, 8), dtype=jnp.float32)
    zeros32 = jnp.zeros((STRIPE, 32), dtype=jnp.float32)

    deg_kernel = _make_deg_kernel()
    agg_kernel = _make_agg_kernel()

    deg_parts = deg_kernel(dst_p, ones8, zeros8)
    d0, d1 = deg_parts[0], deg_parts[1]

    grid = N_PAD // _R
    ts = pl.pallas_call(
        _tc1_body,
        grid=(grid,),
        in_specs=[_row_spec(128), _full_spec((128, 32)), _row_spec(8), _row_spec(8)],
        out_specs=_row_spec(32),
        out_shape=jax.ShapeDtypeStruct((N_PAD, 32), jnp.float32),
    )(x_pad, W1, d0, d1)

    agg1 = agg_kernel(ts, src_p, dst_p, zeros32)

    hs = pl.pallas_call(
        _tc2_body,
        grid=(grid,),
        in_specs=[_row_spec(32), _row_spec(32), _row_spec(32),
                  _row_spec(8), _row_spec(8), _full_spec((1, 32))],
        out_specs=_row_spec(32),
        out_shape=jax.ShapeDtypeStruct((N_PAD, 32), jnp.float32),
    )(agg1[0], agg1[1], ts, d0, d1, b1.reshape(1, 32))

    agg2 = agg_kernel(hs, src_p, dst_p, zeros32)

    mu, ls = pl.pallas_call(
        _tc3_body,
        grid=(grid,),
        in_specs=[_row_spec(32), _row_spec(32), _row_spec(32),
                  _row_spec(8), _row_spec(8),
                  _full_spec((32, 16)), _full_spec((1, 16)),
                  _full_spec((32, 16)), _full_spec((1, 16))],
        out_specs=[_row_spec(16), _row_spec(16)],
        out_shape=[jax.ShapeDtypeStruct((N_PAD, 16), jnp.float32),
                   jax.ShapeDtypeStruct((N_PAD, 16), jnp.float32)],
    )(agg2[0], agg2[1], hs, d0, d1,
      W_mu, b_mu.reshape(1, 16), W_ls, b_ls.reshape(1, 16))

    return (mu[:N_NODES_K], ls[:N_NODES_K])


# speedup vs baseline: 30.3605x; 1.0527x over previous
import functools

import jax
import jax.numpy as jnp
from jax import lax
from jax.experimental import pallas as pl
from jax.experimental.pallas import tpu as pltpu
from jax.experimental.pallas import tpu_sc as plsc

N_NODES_K = 10000
N_PAD = 10240
TRASH = N_NODES_K
NC = 2
NS = 16
NW = NC * NS
CH = 128
E_EDGES = 320000
NBUF = 4
K_PW = 80
E_PAD = NW * CH * K_PW
STRIPE = N_PAD // NS

_MESH = plsc.VectorSubcoreMesh(
    core_axis_name="c", subcore_axis_name="s", num_cores=NC, num_subcores=NS)


def _make_deg_kernel():
    @functools.partial(
        pl.kernel,
        out_type=jax.ShapeDtypeStruct((NC, N_PAD, 8), jnp.float32),
        mesh=_MESH,
        scratch_types=[
            pltpu.VMEM((K_PW, CH), jnp.int32),
            pltpu.VMEM((CH, 8), jnp.float32),
            pltpu.VMEM_SHARED((N_PAD, 8), jnp.float32),
        ],
        compiler_params=pltpu.CompilerParams(use_tc_tiling_on_sc=False),
    )
    def deg_kernel(dst_hbm, ones_hbm, zeros_hbm, out_hbm, dst_v, ones_v, acc):
        cid = lax.axis_index("c")
        sid = lax.axis_index("s")
        wid = cid * NS + sid
        rows = pl.ds(sid * STRIPE, STRIPE)
        pltpu.sync_copy(zeros_hbm, acc.at[rows])
        pltpu.sync_copy(dst_hbm.at[wid], dst_v)
        pltpu.sync_copy(ones_hbm, ones_v)
        plsc.subcore_barrier()

        def body(j, carry):
            pltpu.sync_copy(ones_v, acc.at[dst_v.at[j]], add=True)
            return carry

        lax.fori_loop(0, K_PW, body, 0)
        plsc.subcore_barrier()
        pltpu.sync_copy(acc.at[rows], out_hbm.at[cid, rows])

    return deg_kernel


def _make_agg_kernel():
    @functools.partial(
        pl.kernel,
        out_type=jax.ShapeDtypeStruct((NC, N_PAD, 32), jnp.float32),
        mesh=_MESH,
        scratch_types=[
            pltpu.VMEM((K_PW, CH), jnp.int32),
            pltpu.VMEM((K_PW, CH), jnp.int32),
            [pltpu.VMEM((CH, 32), jnp.float32) for _ in range(NBUF)],
            pltpu.VMEM_SHARED((N_PAD, 32), jnp.float32),
            [pltpu.SemaphoreType.DMA for _ in range(NBUF)],
        ],
        compiler_params=pltpu.CompilerParams(use_tc_tiling_on_sc=False),
    )
    def agg_kernel(table_hbm, src_hbm, dst_hbm, zeros_hbm, out_hbm,
                   src_v, dst_v, bufs, acc, sems):
        cid = lax.axis_index("c")
        sid = lax.axis_index("s")
        wid = cid * NS + sid
        rows = pl.ds(sid * STRIPE, STRIPE)
        pltpu.sync_copy(zeros_hbm, acc.at[rows])
        pltpu.sync_copy(src_hbm.at[wid], src_v)
        pltpu.sync_copy(dst_hbm.at[wid], dst_v)
        plsc.subcore_barrier()

        for b in range(NBUF - 1):
            pltpu.async_copy(table_hbm.at[src_v.at[b]], bufs[b], sems[b])

        def body(i, carry):
            for b in range(NBUF):
                c = i * NBUF + b
                nxt = c + NBUF - 1
                nb = (b + NBUF - 1) % NBUF

                @pl.when(nxt < K_PW)
                def _():
                    pltpu.async_copy(
                        table_hbm.at[src_v.at[nxt]], bufs[nb], sems[nb])

                pltpu.make_async_copy(
                    table_hbm.at[src_v.at[c]], bufs[b], sems[b]).wait()
                pltpu.sync_copy(bufs[b], acc.at[dst_v.at[c]], add=True)
            return carry

        lax.fori_loop(0, K_PW // NBUF, body, 0)
        plsc.subcore_barrier()
        pltpu.sync_copy(acc.at[rows], out_hbm.at[cid, rows])

    return agg_kernel


_R = 2048


def _tc1_body(x_ref, w_ref, d0_ref, d1_ref, out_ref):
    deg = d0_ref[:, 0:1] + d1_ref[:, 0:1] + 1.0
    dinv = lax.rsqrt(deg)
    t = jnp.dot(x_ref[...], w_ref[...], preferred_element_type=jnp.float32)
    out_ref[...] = t * dinv


def _tc2_body(a0_ref, a1_ref, ts_ref, d0_ref, d1_ref, b1_ref, out_ref):
    deg = d0_ref[:, 0:1] + d1_ref[:, 0:1] + 1.0
    dinv = lax.rsqrt(deg)
    h = dinv * (a0_ref[...] + a1_ref[...] + ts_ref[...]) + b1_ref[...]
    out_ref[...] = jnp.maximum(h, 0.0) * dinv


def _tc3_body(a0_ref, a1_ref, hs_ref, d0_ref, d1_ref,
              wm_ref, bm_ref, wl_ref, bl_ref, mu_ref, ls_ref):
    deg = d0_ref[:, 0:1] + d1_ref[:, 0:1] + 1.0
    dinv = lax.rsqrt(deg)
    p = dinv * (a0_ref[...] + a1_ref[...] + hs_ref[...])
    mu_ref[...] = jnp.dot(p, wm_ref[...], preferred_element_type=jnp.float32) + bm_ref[...]
    ls_ref[...] = jnp.dot(p, wl_ref[...], preferred_element_type=jnp.float32) + bl_ref[...]


def _row_spec(w):
    return pl.BlockSpec((_R, w), lambda i: (i, 0))


def _full_spec(shape):
    return pl.BlockSpec(shape, lambda i: (0,) * len(shape))


def kernel(x, edge_index, W1, b1, W_mu, b_mu, W_ls, b_ls):
    src = edge_index[0].astype(jnp.int32)
    dst = edge_index[1].astype(jnp.int32)
    npad_e = E_PAD - E_EDGES
    pad_idx = jnp.full((npad_e,), TRASH, dtype=jnp.int32)
    src_p = jnp.concatenate([src, pad_idx]).reshape(NW, K_PW, CH)
    dst_p = jnp.concatenate([dst, pad_idx]).reshape(NW, K_PW, CH)

    x_pad = jnp.pad(x, ((0, N_PAD - N_NODES_K), (0, 0)))
    ones8 = jnp.ones((CH, 8), dtype=jnp.float32)
    zeros8 = jnp.zeros((STRIPE, 8), dtype=jnp.float32)
    zeros32 = jnp.zeros((STRIPE, 32), dtype=jnp.float32)

    deg_kernel = _make_deg_kernel()
    agg_kernel = _make_agg_kernel()

    deg_parts = deg_kernel(dst_p, ones8, zeros8)
    d0, d1 = deg_parts[0], deg_parts[1]

    grid = N_PAD // _R
    ts = pl.pallas_call(
        _tc1_body,
        grid=(grid,),
        in_specs=[_row_spec(128), _full_spec((128, 32)), _row_spec(8), _row_spec(8)],
        out_specs=_row_spec(32),
        out_shape=jax.ShapeDtypeStruct((N_PAD, 32), jnp.float32),
    )(x_pad, W1, d0, d1)

    agg1 = agg_kernel(ts, src_p, dst_p, zeros32)

    hs = pl.pallas_call(
        _tc2_body,
        grid=(grid,),
        in_specs=[_row_spec(32), _row_spec(32), _row_spec(32),
                  _row_spec(8), _row_spec(8), _full_spec((1, 32))],
        out_specs=_row_spec(32),
        out_shape=jax.ShapeDtypeStruct((N_PAD, 32), jnp.float32),
    )(agg1[0], agg1[1], ts, d0, d1, b1.reshape(1, 32))

    agg2 = agg_kernel(hs, src_p, dst_p, zeros32)

    mu, ls = pl.pallas_call(
        _tc3_body,
        grid=(grid,),
        in_specs=[_row_spec(32), _row_spec(32), _row_spec(32),
                  _row_spec(8), _row_spec(8),
                  _full_spec((32, 16)), _full_spec((1, 16)),
                  _full_spec((32, 16)), _full_spec((1, 16))],
        out_specs=[_row_spec(16), _row_spec(16)],
        out_shape=[jax.ShapeDtypeStruct((N_PAD, 16), jnp.float32),
                   jax.ShapeDtypeStruct((N_PAD, 16), jnp.float32)],
    )(agg2[0], agg2[1], hs, d0, d1,
      W_mu, b_mu.reshape(1, 16), W_ls, b_ls.reshape(1, 16))

    return (mu[:N_NODES_K], ls[:N_NODES_K])


# speedup vs baseline: 30.3741x; 1.0004x over previous
import functools

import jax
import jax.numpy as jnp
from jax import lax
from jax.experimental import pallas as pl
from jax.experimental.pallas import tpu as pltpu
from jax.experimental.pallas import tpu_sc as plsc

N_NODES_K = 10000
N_PAD = 10240
TRASH = N_NODES_K
NC = 2
NS = 16
NW = NC * NS
CH = 128
E_EDGES = 320000
NBUF = 4
K_PW = 80
E_PAD = NW * CH * K_PW
STRIPE = N_PAD // NS

_MESH = plsc.VectorSubcoreMesh(
    core_axis_name="c", subcore_axis_name="s", num_cores=NC, num_subcores=NS)


def _make_deg_kernel():
    @functools.partial(
        pl.kernel,
        out_type=jax.ShapeDtypeStruct((NC, N_PAD, 8), jnp.float32),
        mesh=_MESH,
        scratch_types=[
            pltpu.VMEM((K_PW, CH), jnp.int32),
            pltpu.VMEM((CH, 8), jnp.float32),
            pltpu.VMEM_SHARED((N_PAD, 8), jnp.float32),
        ],
        compiler_params=pltpu.CompilerParams(use_tc_tiling_on_sc=False),
    )
    def deg_kernel(dst_hbm, ones_hbm, zeros_hbm, out_hbm, dst_v, ones_v, acc):
        cid = lax.axis_index("c")
        sid = lax.axis_index("s")
        wid = cid * NS + sid
        rows = pl.ds(sid * STRIPE, STRIPE)
        pltpu.sync_copy(zeros_hbm, acc.at[rows])
        pltpu.sync_copy(dst_hbm.at[wid], dst_v)
        pltpu.sync_copy(ones_hbm, ones_v)
        plsc.subcore_barrier()

        def body(j, carry):
            pltpu.sync_copy(ones_v, acc.at[dst_v.at[j]], add=True)
            return carry

        lax.fori_loop(0, K_PW, body, 0)
        plsc.subcore_barrier()
        pltpu.sync_copy(acc.at[rows], out_hbm.at[cid, rows])

    return deg_kernel


def _make_agg_kernel():
    @functools.partial(
        pl.kernel,
        out_type=jax.ShapeDtypeStruct((NC, N_PAD, 32), jnp.float32),
        mesh=_MESH,
        scratch_types=[
            pltpu.VMEM((K_PW, CH), jnp.int32),
            pltpu.VMEM((K_PW, CH), jnp.int32),
            [pltpu.VMEM((CH, 32), jnp.float32) for _ in range(NBUF)],
            pltpu.VMEM_SHARED((N_PAD, 32), jnp.float32),
            [pltpu.SemaphoreType.DMA for _ in range(NBUF)],
            [pltpu.SemaphoreType.DMA for _ in range(NBUF)],
        ],
        compiler_params=pltpu.CompilerParams(use_tc_tiling_on_sc=False),
    )
    def agg_kernel(table_hbm, src_hbm, dst_hbm, zeros_hbm, out_hbm,
                   src_v, dst_v, bufs, acc, gsems, ssems):
        cid = lax.axis_index("c")
        sid = lax.axis_index("s")
        wid = cid * NS + sid
        rows = pl.ds(sid * STRIPE, STRIPE)
        pltpu.sync_copy(zeros_hbm, acc.at[rows])
        pltpu.sync_copy(src_hbm.at[wid], src_v)
        pltpu.sync_copy(dst_hbm.at[wid], dst_v)
        plsc.subcore_barrier()

        for b in range(NBUF - 1):
            pltpu.async_copy(table_hbm.at[src_v.at[b]], bufs[b], gsems[b])

        def body(i, carry):
            for b in range(NBUF):
                c = i * NBUF + b
                nxt = c + NBUF - 1
                nb = (b + NBUF - 1) % NBUF

                @pl.when((nxt < K_PW) & (c >= 1))
                def _():
                    pltpu.make_async_copy(
                        bufs[nb], acc.at[dst_v.at[c - 1]], ssems[nb]).wait()

                @pl.when(nxt < K_PW)
                def _():
                    pltpu.async_copy(
                        table_hbm.at[src_v.at[nxt]], bufs[nb], gsems[nb])

                pltpu.make_async_copy(
                    table_hbm.at[src_v.at[c]], bufs[b], gsems[b]).wait()
                pltpu.async_copy(
                    bufs[b], acc.at[dst_v.at[c]], ssems[b], add=True)
            return carry

        lax.fori_loop(0, K_PW // NBUF, body, 0)
        for b in range(NBUF):
            pltpu.make_async_copy(
                bufs[b], acc.at[dst_v.at[K_PW - NBUF + b]], ssems[b]).wait()
        plsc.subcore_barrier()
        pltpu.sync_copy(acc.at[rows], out_hbm.at[cid, rows])

    return agg_kernel


_R = 2048


def _tc1_body(x_ref, w_ref, d0_ref, d1_ref, out_ref):
    deg = d0_ref[:, 0:1] + d1_ref[:, 0:1] + 1.0
    dinv = lax.rsqrt(deg)
    t = jnp.dot(x_ref[...], w_ref[...], preferred_element_type=jnp.float32)
    out_ref[...] = t * dinv


def _tc2_body(a0_ref, a1_ref, ts_ref, d0_ref, d1_ref, b1_ref, out_ref):
    deg = d0_ref[:, 0:1] + d1_ref[:, 0:1] + 1.0
    dinv = lax.rsqrt(deg)
    h = dinv * (a0_ref[...] + a1_ref[...] + ts_ref[...]) + b1_ref[...]
    out_ref[...] = jnp.maximum(h, 0.0) * dinv


def _tc3_body(a0_ref, a1_ref, hs_ref, d0_ref, d1_ref,
              wm_ref, bm_ref, wl_ref, bl_ref, mu_ref, ls_ref):
    deg = d0_ref[:, 0:1] + d1_ref[:, 0:1] + 1.0
    dinv = lax.rsqrt(deg)
    p = dinv * (a0_ref[...] + a1_ref[...] + hs_ref[...])
    mu_ref[...] = jnp.dot(p, wm_ref[...], preferred_element_type=jnp.float32) + bm_ref[...]
    ls_ref[...] = jnp.dot(p, wl_ref[...], preferred_element_type=jnp.float32) + bl_ref[...]


def _row_spec(w):
    return pl.BlockSpec((_R, w), lambda i: (i, 0))


def _full_spec(shape):
    return pl.BlockSpec(shape, lambda i: (0,) * len(shape))


def kernel(x, edge_index, W1, b1, W_mu, b_mu, W_ls, b_ls):
    src = edge_index[0].astype(jnp.int32)
    dst = edge_index[1].astype(jnp.int32)
    npad_e = E_PAD - E_EDGES
    pad_idx = jnp.full((npad_e,), TRASH, dtype=jnp.int32)
    src_p = jnp.concatenate([src, pad_idx]).reshape(NW, K_PW, CH)
    dst_p = jnp.concatenate([dst, pad_idx]).reshape(NW, K_PW, CH)

    x_pad = jnp.pad(x, ((0, N_PAD - N_NODES_K), (0, 0)))
    ones8 = jnp.ones((CH, 8), dtype=jnp.float32)
    zeros8 = jnp.zeros((STRIPE, 8), dtype=jnp.float32)
    zeros32 = jnp.zeros((STRIPE, 32), dtype=jnp.float32)

    deg_kernel = _make_deg_kernel()
    agg_kernel = _make_agg_kernel()

    deg_parts = deg_kernel(dst_p, ones8, zeros8)
    d0, d1 = deg_parts[0], deg_parts[1]

    grid = N_PAD // _R
    ts = pl.pallas_call(
        _tc1_body,
        grid=(grid,),
        in_specs=[_row_spec(128), _full_spec((128, 32)), _row_spec(8), _row_spec(8)],
        out_specs=_row_spec(32),
        out_shape=jax.ShapeDtypeStruct((N_PAD, 32), jnp.float32),
    )(x_pad, W1, d0, d1)

    agg1 = agg_kernel(ts, src_p, dst_p, zeros32)

    hs = pl.pallas_call(
        _tc2_body,
        grid=(grid,),
        in_specs=[_row_spec(32), _row_spec(32), _row_spec(32),
                  _row_spec(8), _row_spec(8), _full_spec((1, 32))],
        out_specs=_row_spec(32),
        out_shape=jax.ShapeDtypeStruct((N_PAD, 32), jnp.float32),
    )(agg1[0], agg1[1], ts, d0, d1, b1.reshape(1, 32))

    agg2 = agg_kernel(hs, src_p, dst_p, zeros32)

    mu, ls = pl.pallas_call(
        _tc3_body,
        grid=(grid,),
        in_specs=[_row_spec(32), _row_spec(32), _row_spec(32),
                  _row_spec(8), _row_spec(8),
                  _full_spec((32, 16)), _full_spec((1, 16)),
                  _full_spec((32, 16)), _full_spec((1, 16))],
        out_specs=[_row_spec(16), _row_spec(16)],
        out_shape=[jax.ShapeDtypeStruct((N_PAD, 16), jnp.float32),
                   jax.ShapeDtypeStruct((N_PAD, 16), jnp.float32)],
    )(agg2[0], agg2[1], hs, d0, d1,
      W_mu, b_mu.reshape(1, 16), W_ls, b_ls.reshape(1, 16))

    return (mu[:N_NODES_K], ls[:N_NODES_K])


# speedup vs baseline: 48.4856x; 1.5963x over previous
import functools

import jax
import jax.numpy as jnp
from jax import lax
from jax.experimental import pallas as pl
from jax.experimental.pallas import tpu as pltpu
from jax.experimental.pallas import tpu_sc as plsc

N_NODES_K = 10000
N_PAD = 10240
TRASH = N_NODES_K
NC = 2
NS = 16
NW = NC * NS
CH = 128
E_EDGES = 320000
NBUF = 4
K_PW = 80
E_PAD = NW * CH * K_PW
STRIPE = N_PAD // NS

_MESH = plsc.VectorSubcoreMesh(
    core_axis_name="c", subcore_axis_name="s", num_cores=NC, num_subcores=NS)


def _make_deg_kernel():
    @functools.partial(
        pl.kernel,
        out_type=jax.ShapeDtypeStruct((NC, N_PAD, 8), jnp.float32),
        mesh=_MESH,
        scratch_types=[
            pltpu.VMEM((K_PW, CH), jnp.int32),
            pltpu.VMEM((CH, 8), jnp.float32),
            pltpu.VMEM_SHARED((N_PAD, 8), jnp.float32),
        ],
        compiler_params=pltpu.CompilerParams(use_tc_tiling_on_sc=False),
    )
    def deg_kernel(dst_hbm, ones_hbm, zeros_hbm, out_hbm, dst_v, ones_v, acc):
        cid = lax.axis_index("c")
        sid = lax.axis_index("s")
        wid = cid * NS + sid
        rows = pl.ds(sid * STRIPE, STRIPE)
        pltpu.sync_copy(zeros_hbm, acc.at[rows])
        pltpu.sync_copy(dst_hbm.at[wid], dst_v)
        pltpu.sync_copy(ones_hbm, ones_v)
        plsc.subcore_barrier()

        def body(j, carry):
            pltpu.sync_copy(ones_v, acc.at[dst_v.at[j]], add=True)
            return carry

        lax.fori_loop(0, K_PW, body, 0)
        plsc.subcore_barrier()
        pltpu.sync_copy(acc.at[rows], out_hbm.at[cid, rows])

    return deg_kernel


def _make_agg_kernel():
    @functools.partial(
        pl.kernel,
        out_type=jax.ShapeDtypeStruct((NC, N_PAD, 32), jnp.float32),
        mesh=_MESH,
        scratch_types=[
            pltpu.VMEM((K_PW, CH), jnp.int32),
            pltpu.VMEM((K_PW, CH), jnp.int32),
            [pltpu.VMEM((CH, 32), jnp.float32) for _ in range(NBUF)],
            pltpu.VMEM_SHARED((N_PAD, 32), jnp.float32),
            pltpu.VMEM_SHARED((N_PAD, 32), jnp.float32),
            [pltpu.SemaphoreType.DMA for _ in range(NBUF)],
            [pltpu.SemaphoreType.DMA for _ in range(NBUF)],
        ],
        compiler_params=pltpu.CompilerParams(use_tc_tiling_on_sc=False),
    )
    def agg_kernel(table_hbm, src_hbm, dst_hbm, zeros_hbm, out_hbm,
                   src_v, dst_v, bufs, acc, table_sp, gsems, ssems):
        cid = lax.axis_index("c")
        sid = lax.axis_index("s")
        wid = cid * NS + sid
        rows = pl.ds(sid * STRIPE, STRIPE)
        pltpu.sync_copy(table_hbm.at[rows], table_sp.at[rows])
        pltpu.sync_copy(zeros_hbm, acc.at[rows])
        pltpu.sync_copy(src_hbm.at[wid], src_v)
        pltpu.sync_copy(dst_hbm.at[wid], dst_v)
        plsc.subcore_barrier()

        for b in range(NBUF - 1):
            pltpu.async_copy(table_sp.at[src_v.at[b]], bufs[b], gsems[b])

        def body(i, carry):
            for b in range(NBUF):
                c = i * NBUF + b
                nxt = c + NBUF - 1
                nb = (b + NBUF - 1) % NBUF

                @pl.when((nxt < K_PW) & (c >= 1))
                def _():
                    pltpu.make_async_copy(
                        bufs[nb], acc.at[dst_v.at[c - 1]], ssems[nb]).wait()

                @pl.when(nxt < K_PW)
                def _():
                    pltpu.async_copy(
                        table_sp.at[src_v.at[nxt]], bufs[nb], gsems[nb])

                pltpu.make_async_copy(
                    table_sp.at[src_v.at[c]], bufs[b], gsems[b]).wait()
                pltpu.async_copy(
                    bufs[b], acc.at[dst_v.at[c]], ssems[b], add=True)
            return carry

        lax.fori_loop(0, K_PW // NBUF, body, 0)
        for b in range(NBUF):
            pltpu.make_async_copy(
                bufs[b], acc.at[dst_v.at[K_PW - NBUF + b]], ssems[b]).wait()
        plsc.subcore_barrier()
        pltpu.sync_copy(acc.at[rows], out_hbm.at[cid, rows])

    return agg_kernel


_R = 2048


def _tc1_body(x_ref, w_ref, d0_ref, d1_ref, out_ref):
    deg = d0_ref[:, 0:1] + d1_ref[:, 0:1] + 1.0
    dinv = lax.rsqrt(deg)
    t = jnp.dot(x_ref[...], w_ref[...], preferred_element_type=jnp.float32)
    out_ref[...] = t * dinv


def _tc2_body(a0_ref, a1_ref, ts_ref, d0_ref, d1_ref, b1_ref, out_ref):
    deg = d0_ref[:, 0:1] + d1_ref[:, 0:1] + 1.0
    dinv = lax.rsqrt(deg)
    h = dinv * (a0_ref[...] + a1_ref[...] + ts_ref[...]) + b1_ref[...]
    out_ref[...] = jnp.maximum(h, 0.0) * dinv


def _tc3_body(a0_ref, a1_ref, hs_ref, d0_ref, d1_ref,
              wm_ref, bm_ref, wl_ref, bl_ref, mu_ref, ls_ref):
    deg = d0_ref[:, 0:1] + d1_ref[:, 0:1] + 1.0
    dinv = lax.rsqrt(deg)
    p = dinv * (a0_ref[...] + a1_ref[...] + hs_ref[...])
    mu_ref[...] = jnp.dot(p, wm_ref[...], preferred_element_type=jnp.float32) + bm_ref[...]
    ls_ref[...] = jnp.dot(p, wl_ref[...], preferred_element_type=jnp.float32) + bl_ref[...]


def _row_spec(w):
    return pl.BlockSpec((_R, w), lambda i: (i, 0))


def _full_spec(shape):
    return pl.BlockSpec(shape, lambda i: (0,) * len(shape))


def kernel(x, edge_index, W1, b1, W_mu, b_mu, W_ls, b_ls):
    src = edge_index[0].astype(jnp.int32)
    dst = edge_index[1].astype(jnp.int32)
    npad_e = E_PAD - E_EDGES
    pad_idx = jnp.full((npad_e,), TRASH, dtype=jnp.int32)
    src_p = jnp.concatenate([src, pad_idx]).reshape(NW, K_PW, CH)
    dst_p = jnp.concatenate([dst, pad_idx]).reshape(NW, K_PW, CH)

    x_pad = jnp.pad(x, ((0, N_PAD - N_NODES_K), (0, 0)))
    ones8 = jnp.ones((CH, 8), dtype=jnp.float32)
    zeros8 = jnp.zeros((STRIPE, 8), dtype=jnp.float32)
    zeros32 = jnp.zeros((STRIPE, 32), dtype=jnp.float32)

    deg_kernel = _make_deg_kernel()
    agg_kernel = _make_agg_kernel()

    deg_parts = deg_kernel(dst_p, ones8, zeros8)
    d0, d1 = deg_parts[0], deg_parts[1]

    grid = N_PAD // _R
    ts = pl.pallas_call(
        _tc1_body,
        grid=(grid,),
        in_specs=[_row_spec(128), _full_spec((128, 32)), _row_spec(8), _row_spec(8)],
        out_specs=_row_spec(32),
        out_shape=jax.ShapeDtypeStruct((N_PAD, 32), jnp.float32),
    )(x_pad, W1, d0, d1)

    agg1 = agg_kernel(ts, src_p, dst_p, zeros32)

    hs = pl.pallas_call(
        _tc2_body,
        grid=(grid,),
        in_specs=[_row_spec(32), _row_spec(32), _row_spec(32),
                  _row_spec(8), _row_spec(8), _full_spec((1, 32))],
        out_specs=_row_spec(32),
        out_shape=jax.ShapeDtypeStruct((N_PAD, 32), jnp.float32),
    )(agg1[0], agg1[1], ts, d0, d1, b1.reshape(1, 32))

    agg2 = agg_kernel(hs, src_p, dst_p, zeros32)

    mu, ls = pl.pallas_call(
        _tc3_body,
        grid=(grid,),
        in_specs=[_row_spec(32), _row_spec(32), _row_spec(32),
                  _row_spec(8), _row_spec(8),
                  _full_spec((32, 16)), _full_spec((1, 16)),
                  _full_spec((32, 16)), _full_spec((1, 16))],
        out_specs=[_row_spec(16), _row_spec(16)],
        out_shape=[jax.ShapeDtypeStruct((N_PAD, 16), jnp.float32),
                   jax.ShapeDtypeStruct((N_PAD, 16), jnp.float32)],
    )(agg2[0], agg2[1], hs, d0, d1,
      W_mu, b_mu.reshape(1, 16), W_ls, b_ls.reshape(1, 16))

    return (mu[:N_NODES_K], ls[:N_NODES_K])
